# Initial kernel scaffold; baseline (speedup 1.0000x reference)
import jax
import jax.numpy as jnp
from jax.experimental import pallas as pl


def kernel(input, adj, W, b, attn_w):
    raise NotImplementedError("write your pallas kernel here")


# speedup vs baseline: 2.3291x; 2.3291x over previous
import functools

import jax
import jax.numpy as jnp
from jax.experimental import pallas as pl
from jax.experimental.pallas import tpu as pltpu

_N = 10000
_F = 128
_ALPHA = 0.2

_NP = 10240
_BR = 512
_BC = 1024
_PR = 1024


def _project_kernel(x_ref, w_ref, b_ref, aw_ref, out_ref, st_ref):
    o = jnp.dot(x_ref[...], w_ref[...], preferred_element_type=jnp.float32)
    o = o + b_ref[...]
    out_ref[...] = o
    st_ref[...] = jnp.dot(o, aw_ref[...], preferred_element_type=jnp.float32)


def _gat_kernel(adj_ref, s_ref, t_ref, out_ref, y_ref, acc_ref, rs_ref,
                *, nj):
    j = pl.program_id(1)

    @pl.when(j == 0)
    def _init():
        acc_ref[...] = jnp.zeros_like(acc_ref)
        rs_ref[...] = jnp.zeros_like(rs_ref)

    t = t_ref[:, pl.ds(j * _BC, _BC)]
    e = s_ref[...] + t
    e = jnp.where(e > 0, e, _ALPHA * e)
    ev = jnp.exp(e) * adj_ref[...]
    col = j * _BC + jax.lax.broadcasted_iota(jnp.int32, (_BR, _BC), 1)
    ev = jnp.where(col < _N, ev, 0.0)

    rs_ref[...] += jnp.sum(ev, axis=1, keepdims=True)
    acc_ref[...] += jnp.dot(ev, out_ref[pl.ds(j * _BC, _BC), :],
                            preferred_element_type=jnp.float32)

    @pl.when(j == nj - 1)
    def _finish():
        rs = rs_ref[...]
        y_ref[...] = acc_ref[...] / jnp.where(rs == 0.0, 1.0, rs)


def kernel(input, adj, W, b, attn_w):
    x = jnp.zeros((_NP, _F), jnp.float32).at[:_N].set(input)
    aw = attn_w.reshape(_F, 2)
    b2 = b.reshape(1, _F)

    ni_p = _NP // _PR
    out, st = pl.pallas_call(
        _project_kernel,
        grid=(ni_p,),
        in_specs=[
            pl.BlockSpec((_PR, _F), lambda i: (i, 0)),
            pl.BlockSpec((_F, _F), lambda i: (0, 0)),
            pl.BlockSpec((1, _F), lambda i: (0, 0)),
            pl.BlockSpec((_F, 2), lambda i: (0, 0)),
        ],
        out_specs=[
            pl.BlockSpec((_PR, _F), lambda i: (i, 0)),
            pl.BlockSpec((_PR, 2), lambda i: (i, 0)),
        ],
        out_shape=[
            jax.ShapeDtypeStruct((_NP, _F), jnp.float32),
            jax.ShapeDtypeStruct((_NP, 2), jnp.float32),
        ],
    )(x, W, b2, aw)

    s = st[:, 0:1]
    t = st[:, 1:2].T

    ni, nj = _NP // _BR, _NP // _BC
    y = pl.pallas_call(
        functools.partial(_gat_kernel, nj=nj),
        grid=(ni, nj),
        in_specs=[
            pl.BlockSpec((_BR, _BC), lambda i, j: (i, j)),
            pl.BlockSpec((_BR, 1), lambda i, j: (i, 0)),
            pl.BlockSpec((1, _NP), lambda i, j: (0, 0)),
            pl.BlockSpec((_NP, _F), lambda i, j: (0, 0)),
        ],
        out_specs=pl.BlockSpec((_BR, _F), lambda i, j: (i, 0)),
        out_shape=jax.ShapeDtypeStruct((_NP, _F), jnp.float32),
        scratch_shapes=[
            pltpu.VMEM((_BR, _F), jnp.float32),
            pltpu.VMEM((_BR, 1), jnp.float32),
        ],
    )(adj, s, t, out)

    return y[:_N]


# speedup vs baseline: 2.4230x; 1.0403x over previous
import functools

import jax
import jax.numpy as jnp
from jax.experimental import pallas as pl
from jax.experimental.pallas import tpu as pltpu

_N = 10000
_F = 128
_ALPHA = 0.2

_NP = 10240
_BR = 512
_BC = 1024
_PR = 512


def _project_kernel(x_ref, w_ref, b_ref, aw_ref, out_ref, st_ref):
    i = pl.program_id(0)
    o = jnp.dot(x_ref[...], w_ref[...], preferred_element_type=jnp.float32)
    o = o + b_ref[...]
    row = i * _PR + jax.lax.broadcasted_iota(jnp.int32, (_PR, 1), 0)
    o = jnp.where(row < _N, o, 0.0)
    out_ref[...] = o
    st_ref[...] = jnp.dot(o, aw_ref[...], preferred_element_type=jnp.float32)


def _gat_kernel(adj_ref, s_ref, t_ref, out_ref, y_ref, acc_ref, rs_ref,
                *, nj):
    j = pl.program_id(1)

    @pl.when(j == 0)
    def _init():
        acc_ref[...] = jnp.zeros_like(acc_ref)
        rs_ref[...] = jnp.zeros_like(rs_ref)

    e = s_ref[...] + t_ref[:, pl.ds(j * _BC, _BC)]
    e = jnp.maximum(e, _ALPHA * e)
    ev = jnp.exp(e) * adj_ref[...]

    @pl.when(j < nj - 1)
    def _acc_body():
        rs_ref[...] += jnp.sum(ev, axis=1, keepdims=True)
        acc_ref[...] += jnp.dot(ev, out_ref[pl.ds(j * _BC, _BC), :],
                                preferred_element_type=jnp.float32)

    @pl.when(j == nj - 1)
    def _acc_last():
        col = j * _BC + jax.lax.broadcasted_iota(jnp.int32, (_BR, _BC), 1)
        evm = jnp.where(col < _N, ev, 0.0)
        rs = rs_ref[...] + jnp.sum(evm, axis=1, keepdims=True)
        acc = acc_ref[...] + jnp.dot(evm, out_ref[pl.ds(j * _BC, _BC), :],
                                     preferred_element_type=jnp.float32)
        y_ref[...] = acc / jnp.where(rs == 0.0, 1.0, rs)


def kernel(input, adj, W, b, attn_w):
    aw = attn_w.reshape(_F, 2)
    b2 = b.reshape(1, _F)

    out, st = pl.pallas_call(
        _project_kernel,
        grid=(_NP // _PR,),
        in_specs=[
            pl.BlockSpec((_PR, _F), lambda i: (i, 0)),
            pl.BlockSpec((_F, _F), lambda i: (0, 0)),
            pl.BlockSpec((1, _F), lambda i: (0, 0)),
            pl.BlockSpec((_F, 2), lambda i: (0, 0)),
        ],
        out_specs=[
            pl.BlockSpec((_PR, _F), lambda i: (i, 0)),
            pl.BlockSpec((_PR, 2), lambda i: (i, 0)),
        ],
        out_shape=[
            jax.ShapeDtypeStruct((_NP, _F), jnp.float32),
            jax.ShapeDtypeStruct((_NP, 2), jnp.float32),
        ],
    )(input, W, b2, aw)

    s = st[:, 0:1]
    t = st[:, 1:2].T

    ni, nj = _NP // _BR, _NP // _BC
    y = pl.pallas_call(
        functools.partial(_gat_kernel, nj=nj),
        grid=(ni, nj),
        in_specs=[
            pl.BlockSpec((_BR, _BC), lambda i, j: (i, j)),
            pl.BlockSpec((_BR, 1), lambda i, j: (i, 0)),
            pl.BlockSpec((1, _NP), lambda i, j: (0, 0)),
            pl.BlockSpec((_NP, _F), lambda i, j: (0, 0)),
        ],
        out_specs=pl.BlockSpec((_BR, _F), lambda i, j: (i, 0)),
        out_shape=jax.ShapeDtypeStruct((_N, _F), jnp.float32),
        scratch_shapes=[
            pltpu.VMEM((_BR, _F), jnp.float32),
            pltpu.VMEM((_BR, 1), jnp.float32),
        ],
    )(adj, s, t, out)

    return y


# speedup vs baseline: 2.9012x; 1.1974x over previous
import functools

import jax
import jax.numpy as jnp
from jax.experimental import pallas as pl
from jax.experimental.pallas import tpu as pltpu

_N = 10000
_F = 128
_ALPHA = 0.2

_NP = 10240
_BR = 1024
_BC = 2560
_PR = 512


def _project_kernel(x_ref, w_ref, b_ref, aw_ref, out_ref, st_ref):
    i = pl.program_id(0)
    o = jnp.dot(x_ref[...], w_ref[...], preferred_element_type=jnp.float32)
    o = o + b_ref[...]
    row = i * _PR + jax.lax.broadcasted_iota(jnp.int32, (_PR, 1), 0)
    o = jnp.where(row < _N, o, 0.0)
    out_ref[...] = o
    st_ref[...] = jnp.dot(o, aw_ref[...], preferred_element_type=jnp.float32)


def _gat_kernel(adj_ref, s_ref, t_ref, out_ref, y_ref, acc_ref, rs_ref,
                *, nj):
    j = pl.program_id(1)

    @pl.when(j == 0)
    def _init():
        acc_ref[...] = jnp.zeros_like(acc_ref)
        rs_ref[...] = jnp.zeros_like(rs_ref)

    e = s_ref[...] + t_ref[:, pl.ds(j * _BC, _BC)]
    e = jnp.maximum(e, _ALPHA * e)
    ev = jnp.exp(e) * adj_ref[...]

    @pl.when(j < nj - 1)
    def _acc_body():
        rs_ref[...] += jnp.sum(ev, axis=1, keepdims=True)
        acc_ref[...] += jnp.dot(ev, out_ref[pl.ds(j * _BC, _BC), :],
                                preferred_element_type=jnp.float32)

    @pl.when(j == nj - 1)
    def _acc_last():
        col = j * _BC + jax.lax.broadcasted_iota(jnp.int32, (_BR, _BC), 1)
        evm = jnp.where(col < _N, ev, 0.0)
        rs = rs_ref[...] + jnp.sum(evm, axis=1, keepdims=True)
        acc = acc_ref[...] + jnp.dot(evm, out_ref[pl.ds(j * _BC, _BC), :],
                                     preferred_element_type=jnp.float32)
        y_ref[...] = acc / jnp.where(rs == 0.0, 1.0, rs)


def kernel(input, adj, W, b, attn_w):
    aw = attn_w.reshape(_F, 2)
    b2 = b.reshape(1, _F)

    out, st = pl.pallas_call(
        _project_kernel,
        grid=(_NP // _PR,),
        in_specs=[
            pl.BlockSpec((_PR, _F), lambda i: (i, 0)),
            pl.BlockSpec((_F, _F), lambda i: (0, 0)),
            pl.BlockSpec((1, _F), lambda i: (0, 0)),
            pl.BlockSpec((_F, 2), lambda i: (0, 0)),
        ],
        out_specs=[
            pl.BlockSpec((_PR, _F), lambda i: (i, 0)),
            pl.BlockSpec((_PR, 2), lambda i: (i, 0)),
        ],
        out_shape=[
            jax.ShapeDtypeStruct((_NP, _F), jnp.float32),
            jax.ShapeDtypeStruct((_NP, 2), jnp.float32),
        ],
    )(input, W, b2, aw)

    s = st[:, 0:1]
    t = st[:, 1:2].T

    ni, nj = _NP // _BR, _NP // _BC
    y = pl.pallas_call(
        functools.partial(_gat_kernel, nj=nj),
        grid=(ni, nj),
        in_specs=[
            pl.BlockSpec((_BR, _BC), lambda i, j: (i, j)),
            pl.BlockSpec((_BR, 1), lambda i, j: (i, 0)),
            pl.BlockSpec((1, _NP), lambda i, j: (0, 0)),
            pl.BlockSpec((_NP, _F), lambda i, j: (0, 0)),
        ],
        out_specs=pl.BlockSpec((_BR, _F), lambda i, j: (i, 0)),
        out_shape=jax.ShapeDtypeStruct((_N, _F), jnp.float32),
        scratch_shapes=[
            pltpu.VMEM((_BR, _F), jnp.float32),
            pltpu.VMEM((_BR, 1), jnp.float32),
        ],
    )(adj, s, t, out)

    return y


# speedup vs baseline: 3.6598x; 1.2615x over previous
import functools

import jax
import jax.numpy as jnp
import numpy as np
from jax.experimental import pallas as pl
from jax.experimental.pallas import tpu as pltpu

_N = 10000
_F = 128
_ALPHA = 0.2

_NP = 10240
_BR = 1024
_BC = 2560
_PR = 512


def _project_kernel(x_ref, w_ref, b_ref, aw_ref, out_ref, st_ref):
    i = pl.program_id(0)
    o = jnp.dot(x_ref[...], w_ref[...], preferred_element_type=jnp.float32)
    o = o + b_ref[...]
    row = i * _PR + jax.lax.broadcasted_iota(jnp.int32, (_PR, 1), 0)
    o = jnp.where(row < _N, o, 0.0)
    out_ref[...] = o
    st_ref[...] = jnp.dot(o, aw_ref[...], preferred_element_type=jnp.float32)


def _gat_kernel(adj_ref, s_ref, t_ref, out_ref, y_ref, acc_ref, rs_ref,
                *, nj):
    j = pl.program_id(1)

    @pl.when(j == 0)
    def _init():
        acc_ref[...] = jnp.zeros_like(acc_ref)
        rs_ref[...] = jnp.zeros_like(rs_ref)

    e = s_ref[...] + t_ref[:, pl.ds(j * _BC, _BC)]
    e = jnp.maximum(e, _ALPHA * e)
    ev = jnp.exp2(e) * adj_ref[...]

    @pl.when(j < nj - 1)
    def _acc_body():
        rs_ref[...] += jnp.sum(ev, axis=1, keepdims=True)
        acc_ref[...] += jnp.dot(ev, out_ref[pl.ds(j * _BC, _BC), :],
                                preferred_element_type=jnp.float32)

    @pl.when(j == nj - 1)
    def _acc_last():
        col = j * _BC + jax.lax.broadcasted_iota(jnp.int32, (_BR, _BC), 1)
        evm = jnp.where(col < _N, ev, 0.0)
        rs = rs_ref[...] + jnp.sum(evm, axis=1, keepdims=True)
        acc = acc_ref[...] + jnp.dot(evm, out_ref[pl.ds(j * _BC, _BC), :],
                                     preferred_element_type=jnp.float32)
        y_ref[...] = acc / jnp.where(rs == 0.0, 1.0, rs)


def kernel(input, adj, W, b, attn_w):
    aw = attn_w.reshape(_F, 2) * np.float32(np.log2(np.e))
    b2 = b.reshape(1, _F)

    out, st = pl.pallas_call(
        _project_kernel,
        grid=(_NP // _PR,),
        in_specs=[
            pl.BlockSpec((_PR, _F), lambda i: (i, 0)),
            pl.BlockSpec((_F, _F), lambda i: (0, 0)),
            pl.BlockSpec((1, _F), lambda i: (0, 0)),
            pl.BlockSpec((_F, 2), lambda i: (0, 0)),
        ],
        out_specs=[
            pl.BlockSpec((_PR, _F), lambda i: (i, 0)),
            pl.BlockSpec((_PR, 2), lambda i: (i, 0)),
        ],
        out_shape=[
            jax.ShapeDtypeStruct((_NP, _F), jnp.float32),
            jax.ShapeDtypeStruct((_NP, 2), jnp.float32),
        ],
    )(input, W, b2, aw)

    s = st[:, 0:1]
    t = st[:, 1:2].T

    ni, nj = _NP // _BR, _NP // _BC
    y = pl.pallas_call(
        functools.partial(_gat_kernel, nj=nj),
        grid=(ni, nj),
        in_specs=[
            pl.BlockSpec((_BR, _BC), lambda i, j: (i, j)),
            pl.BlockSpec((_BR, 1), lambda i, j: (i, 0)),
            pl.BlockSpec((1, _NP), lambda i, j: (0, 0)),
            pl.BlockSpec((_NP, _F), lambda i, j: (0, 0)),
        ],
        out_specs=pl.BlockSpec((_BR, _F), lambda i, j: (i, 0)),
        out_shape=jax.ShapeDtypeStruct((_N, _F), jnp.float32),
        scratch_shapes=[
            pltpu.VMEM((_BR, _F), jnp.float32),
            pltpu.VMEM((_BR, 1), jnp.float32),
        ],
    )(adj, s, t, out)

    return y


# speedup vs baseline: 3.7977x; 1.0377x over previous
import functools

import jax
import jax.numpy as jnp
import numpy as np
from jax.experimental import pallas as pl
from jax.experimental.pallas import tpu as pltpu

_N = 10000
_F = 128
_ALPHA = 0.2

_NP = 10240
_BR = 1024
_BC = 2560
_PR = 512


def _project_kernel(x_ref, w_ref, b_ref, aw_ref, out_ref, st_ref):
    i = pl.program_id(0)
    o = jnp.dot(x_ref[...], w_ref[...], preferred_element_type=jnp.float32)
    o = o + b_ref[...]
    row = i * _PR + jax.lax.broadcasted_iota(jnp.int32, (_PR, 1), 0)
    o = jnp.where(row < _N, o, 0.0)
    col = jax.lax.broadcasted_iota(jnp.int32, (_PR, 2 * _F), 1)
    out_ref[...] = jnp.where(col < _F,
                             jnp.pad(o, ((0, 0), (0, _F))),
                             jnp.where(col == _F, 1.0, 0.0))
    st_ref[...] = jnp.dot(o, aw_ref[...], preferred_element_type=jnp.float32)


def _gat_kernel(adj_ref, s_ref, t_ref, out_ref, y_ref, acc_ref, *, nj):
    j = pl.program_id(1)

    @pl.when(j == 0)
    def _init():
        acc_ref[...] = jnp.zeros_like(acc_ref)

    e = s_ref[...] + t_ref[:, pl.ds(j * _BC, _BC)]
    e = jnp.maximum(e, _ALPHA * e)
    ev = jnp.exp2(e) * adj_ref[...]

    @pl.when(j < nj - 1)
    def _acc_body():
        acc_ref[...] += jnp.dot(ev, out_ref[pl.ds(j * _BC, _BC), :],
                                preferred_element_type=jnp.float32)

    @pl.when(j == nj - 1)
    def _acc_last():
        col = j * _BC + jax.lax.broadcasted_iota(jnp.int32, (_BR, _BC), 1)
        evm = jnp.where(col < _N, ev, 0.0)
        acc = acc_ref[...] + jnp.dot(evm, out_ref[pl.ds(j * _BC, _BC), :],
                                     preferred_element_type=jnp.float32)
        rs = acc[:, _F:_F + 1]
        y_ref[...] = acc[:, :_F] / jnp.where(rs == 0.0, 1.0, rs)


def kernel(input, adj, W, b, attn_w):
    aw = attn_w.reshape(_F, 2) * np.float32(np.log2(np.e))
    b2 = b.reshape(1, _F)

    out, st = pl.pallas_call(
        _project_kernel,
        grid=(_NP // _PR,),
        in_specs=[
            pl.BlockSpec((_PR, _F), lambda i: (i, 0)),
            pl.BlockSpec((_F, _F), lambda i: (0, 0)),
            pl.BlockSpec((1, _F), lambda i: (0, 0)),
            pl.BlockSpec((_F, 2), lambda i: (0, 0)),
        ],
        out_specs=[
            pl.BlockSpec((_PR, 2 * _F), lambda i: (i, 0)),
            pl.BlockSpec((_PR, 2), lambda i: (i, 0)),
        ],
        out_shape=[
            jax.ShapeDtypeStruct((_NP, 2 * _F), jnp.float32),
            jax.ShapeDtypeStruct((_NP, 2), jnp.float32),
        ],
    )(input, W, b2, aw)

    s = st[:, 0:1]
    t = st[:, 1:2].T

    ni, nj = _NP // _BR, _NP // _BC
    y = pl.pallas_call(
        functools.partial(_gat_kernel, nj=nj),
        grid=(ni, nj),
        in_specs=[
            pl.BlockSpec((_BR, _BC), lambda i, j: (i, j)),
            pl.BlockSpec((_BR, 1), lambda i, j: (i, 0)),
            pl.BlockSpec((1, _NP), lambda i, j: (0, 0)),
            pl.BlockSpec((_NP, 2 * _F), lambda i, j: (0, 0)),
        ],
        out_specs=pl.BlockSpec((_BR, _F), lambda i, j: (i, 0)),
        out_shape=jax.ShapeDtypeStruct((_N, _F), jnp.float32),
        scratch_shapes=[
            pltpu.VMEM((_BR, 2 * _F), jnp.float32),
        ],
    )(adj, s, t, out)

    return y
